# baseline (speedup 1.0000x reference)
import functools

import jax
import jax.numpy as jnp
from jax import lax
from jax.experimental import pallas as pl
from jax.experimental.pallas import tpu as pltpu
from jax.experimental.pallas import tpu_sc as plsc

N_CLS = 1000
D = 128
NCLS_PAD = 1024
EMA = 0.99
ONE_M = 0.01
INV_T = 10.0
NW = 32
CPT = NCLS_PAD // NW
NS = 4
CPS = CPT // NS
RC = 64
SGC = 512
NBUF = 34
HPAD = 2048
NQ = D // 16


def _rsqrt16(sv):
    i = lax.bitcast_convert_type(sv, jnp.int32)
    i = jnp.int32(0x5F3759DF) - lax.shift_right_logical(i, 1)
    y = lax.bitcast_convert_type(i, jnp.float32)
    for _ in range(3):
        y = y * (1.5 - 0.5 * sv * y * y)
    return y


def _sc_body(feat, sidx, slbl, protos_in, meta, protos_out,
             meta_v, idx_buf, lbl_buf, rows_v, p0_v, p1_v, p2_v, p3_v,
             stg_sem, sem0, sem1):
    cid = lax.axis_index("c")
    sid = lax.axis_index("s")
    wid = sid * 2 + cid
    base_cls = wid * CPT
    protos_v = [p0_v, p1_v, p2_v, p3_v]

    pltpu.sync_copy(meta.at[wid], meta_v)
    mv = meta_v[...]
    stage_a0 = mv[0]
    nstage = mv[1]
    npair = mv[2]
    a0b = [mv[4 + p] for p in range(NS)]
    skip = [mv[8 + p] for p in range(NS)]
    nrow = [mv[12 + p] for p in range(NS)]
    end = [skip[p] + nrow[p] for p in range(NS)]
    nch = [lax.max((skip[p] + nrow[p] + (RC - 1)) // RC, 1)
           for p in range(NS)]

    for p in range(NS):
        pltpu.sync_copy(protos_in.at[pl.ds(base_cls + p * CPS, CPS)],
                        protos_v[p].at[pl.ds(0, CPS)])
        for q in range(NQ):
            protos_v[p][CPS, pl.ds(q * 16, 16)] = jnp.zeros((16,),
                                                            jnp.float32)

    def stage_fire(s, carry):
        off = pl.multiple_of(stage_a0 + s * SGC, 8)
        boff = pl.multiple_of(s * SGC, 8)
        pltpu.async_copy(sidx.at[pl.ds(off, SGC)],
                         idx_buf.at[pl.ds(boff, SGC)], stg_sem)
        pltpu.async_copy(slbl.at[pl.ds(off, SGC)],
                         lbl_buf.at[pl.ds(boff, SGC)], stg_sem)
        return carry

    lax.fori_loop(0, nstage, stage_fire, 0)

    def stage_drain(s, carry):
        boff = pl.multiple_of(s * SGC, 8)
        pltpu.make_async_copy(sidx.at[pl.ds(0, SGC)],
                              idx_buf.at[pl.ds(boff, SGC)], stg_sem).wait()
        pltpu.make_async_copy(slbl.at[pl.ds(0, SGC)],
                              lbl_buf.at[pl.ds(boff, SGC)], stg_sem).wait()
        return carry

    lax.fori_loop(0, nstage, stage_drain, 0)

    def chunk_off(c, p):
        kk = lax.clamp(jnp.int32(0), c, nch[p] - 1)
        return pl.multiple_of(a0b[p] + kk * RC, 8)

    sems = [sem0, sem1]

    def fire(c, b):
        for p in range(NS):
            pltpu.async_copy(feat.at[idx_buf.at[pl.ds(chunk_off(c, p), RC)]],
                             rows_v.at[b, p], sems[b])

    def drain(b):
        for p in range(NS):
            pltpu.make_async_copy(feat.at[idx_buf.at[pl.ds(0, RC)]],
                                  rows_v.at[b, p], sems[b]).wait()

    def compute(c, b):
        offs = [chunk_off(c, p) for p in range(NS)]

        def grp(g, carry):
            lblv = [lbl_buf[pl.ds(offs[p] + g * 16, 16)] for p in range(NS)]
            for u in range(16):
                i = g * 16 + u
                for p in range(NS):
                    j = c * RC + i
                    valid = jnp.logical_and(j >= skip[p], j < end[p])
                    cc = lax.select(valid,
                                    lblv[p][u] - (base_cls + p * CPS),
                                    jnp.int32(CPS))
                    rs = []
                    sq = []
                    for q in range(NQ):
                        f = rows_v[b, p, i, pl.ds(q * 16, 16)]
                        pv = protos_v[p][cc, pl.ds(q * 16, 16)]
                        r = pv * EMA + f * ONE_M
                        rs.append(r)
                        sq.append(r * r)
                    s01 = (sq[0] + sq[1]) + (sq[2] + sq[3])
                    s23 = (sq[4] + sq[5]) + (sq[6] + sq[7])
                    s = jnp.sum(s01 + s23)
                    sv = jnp.full((16,), s, dtype=jnp.float32)
                    y = jnp.minimum(_rsqrt16(sv), 1e12)
                    for q in range(NQ):
                        protos_v[p][cc, pl.ds(q * 16, 16)] = rs[q] * y
            return carry

        lax.fori_loop(0, RC // 16, grp, 0)

    fire(jnp.int32(0), 0)

    def pair(i, carry):
        for b in range(2):
            c = i * 2 + b
            fire(c + 1, 1 - b)
            drain(b)
            compute(c, b)
        return carry

    lax.fori_loop(0, npair, pair, 0)
    drain(0)

    for p in range(NS):
        pltpu.sync_copy(protos_v[p].at[pl.ds(0, CPS)],
                        protos_out.at[pl.ds(base_cls + p * CPS, CPS)])


_sc_update = functools.partial(
    pl.kernel,
    mesh=plsc.VectorSubcoreMesh(core_axis_name="c", subcore_axis_name="s"),
    out_type=jax.ShapeDtypeStruct((NCLS_PAD, D), jnp.float32),
    scratch_types=[
        pltpu.VMEM((16,), jnp.int32),
        pltpu.VMEM((NBUF * SGC,), jnp.int32),
        pltpu.VMEM((NBUF * SGC,), jnp.int32),
        pltpu.VMEM((2, NS, RC, D), jnp.float32),
        pltpu.VMEM((CPS + 1, D), jnp.float32),
        pltpu.VMEM((CPS + 1, D), jnp.float32),
        pltpu.VMEM((CPS + 1, D), jnp.float32),
        pltpu.VMEM((CPS + 1, D), jnp.float32),
        pltpu.SemaphoreType.DMA,
        pltpu.SemaphoreType.DMA,
        pltpu.SemaphoreType.DMA,
    ],
    compiler_params=pltpu.CompilerParams(needs_layout_passes=False),
)(_sc_body)


def _loss_body(protos_ref, out_ref):
    p = protos_ref[...]
    logits = lax.dot_general(p, p, (((1,), (1,)), ((), ())),
                             preferred_element_type=jnp.float32) * INV_T
    row = lax.broadcasted_iota(jnp.int32, (NCLS_PAD, NCLS_PAD), 0)
    col = lax.broadcasted_iota(jnp.int32, (NCLS_PAD, NCLS_PAD), 1)
    mask = jnp.logical_and(row != col,
                           jnp.logical_and(row < N_CLS, col < N_CLS))
    e = jnp.where(mask, jnp.exp(logits), 0.0)
    ssum = jnp.sum(e, axis=1, keepdims=True)
    mpn = jnp.log(ssum * (1.0 / (N_CLS - 1)))
    rvalid = lax.broadcasted_iota(jnp.int32, (NCLS_PAD, 1), 0) < N_CLS
    tot = jnp.sum(jnp.where(rvalid, mpn, 0.0), axis=0, keepdims=True)
    out_ref[...] = tot * (1.0 / N_CLS)


_loss_call = pl.pallas_call(
    _loss_body,
    out_shape=jax.ShapeDtypeStruct((1, 1), jnp.float32),
)


def kernel(features, labels, prototypes):
    labels = labels.astype(jnp.int32)
    nb = labels.shape[0]
    key = (labels << 14) | jnp.arange(nb, dtype=jnp.int32)
    skey = jnp.sort(key)
    order = skey & jnp.int32(0x3FFF)
    slbl = skey >> 14
    counts = jnp.zeros((NCLS_PAD,), jnp.int32).at[labels].add(1)
    starts = jnp.concatenate(
        [jnp.zeros((1,), jnp.int32), jnp.cumsum(counts, dtype=jnp.int32)])

    w = jnp.arange(NW, dtype=jnp.int32)
    s0t = starts[w * CPT]
    s1t = starts[w * CPT + CPT]
    stage_a0 = (s0t // 8) * 8
    stage_total = s1t - stage_a0
    nstage = (stage_total + SGC - 1) // SGC + 1

    sp = starts[(w[:, None] * CPT
                 + jnp.arange(NS + 1, dtype=jnp.int32)[None, :] * CPS)]
    rel0 = sp[:, :NS] - stage_a0[:, None]
    a0b = (rel0 // 8) * 8
    skip = rel0 - a0b
    nrow = sp[:, 1:] - sp[:, :NS]
    nch = jnp.maximum((skip + nrow + (RC - 1)) // RC, 1)
    npair = (jnp.max(nch, axis=1) + 1) // 2

    meta = jnp.zeros((NW, 16), jnp.int32)
    meta = (meta.at[:, 0].set(stage_a0).at[:, 1].set(nstage)
                .at[:, 2].set(npair)
                .at[:, 4:8].set(a0b).at[:, 8:12].set(skip)
                .at[:, 12:16].set(nrow))

    pad_i = jnp.zeros((HPAD,), jnp.int32)
    sidx_pad = jnp.concatenate([order, pad_i])
    slbl_pad = jnp.concatenate([slbl, pad_i])
    protos_pad = jnp.concatenate(
        [prototypes.astype(jnp.float32),
         jnp.zeros((NCLS_PAD - N_CLS, D), jnp.float32)], axis=0)

    protos_upd = _sc_update(features.astype(jnp.float32), sidx_pad, slbl_pad,
                            protos_pad, meta)
    return _loss_call(protos_upd)[0, 0]


# speedup vs baseline: 1.4605x; 1.4605x over previous
import functools

import jax
import jax.numpy as jnp
from jax import lax
from jax.experimental import pallas as pl
from jax.experimental.pallas import tpu as pltpu
from jax.experimental.pallas import tpu_sc as plsc

N_CLS = 1000
D = 128
NCLS_PAD = 1024
EMA = 0.99
ONE_M = 0.01
INV_T = 10.0
NW = 32
CPT = NCLS_PAD // NW
NS = 2
CPS = CPT // NS
RC = 128
HPAD = 256
NQ = D // 16


def _rsqrt16(sv):
    i = lax.bitcast_convert_type(sv, jnp.int32)
    i = jnp.int32(0x5F3759DF) - lax.shift_right_logical(i, 1)
    y = lax.bitcast_convert_type(i, jnp.float32)
    for _ in range(3):
        y = y * (1.5 - 0.5 * sv * y * y)
    return y


def _sc_body(feat, sidx, slbl, protos_in, meta, protos_out,
             meta_v, idx_v, lbl_v, rows_v, p0_v, p1_v, sem):
    cid = lax.axis_index("c")
    sid = lax.axis_index("s")
    wid = sid * 2 + cid
    base_cls = wid * CPT
    protos_v = [p0_v, p1_v]

    pltpu.sync_copy(meta.at[wid], meta_v)
    mv = meta_v[...]
    a0 = [mv[0 + p] for p in range(NS)]
    skip = [mv[2 + p] for p in range(NS)]
    nrow = [mv[4 + p] for p in range(NS)]
    nch = [mv[6 + p] for p in range(NS)]
    nch_max = mv[8]
    end = [skip[p] + nrow[p] for p in range(NS)]

    for p in range(NS):
        pltpu.sync_copy(protos_in.at[pl.ds(base_cls + p * CPS, CPS)],
                        protos_v[p].at[pl.ds(0, CPS)])
        for q in range(NQ):
            protos_v[p][CPS, pl.ds(q * 16, 16)] = jnp.zeros((16,),
                                                            jnp.float32)

    def chunk_body(k, carry):
        offs = []
        for p in range(NS):
            kk = lax.clamp(jnp.int32(0), k, nch[p] - 1)
            off = pl.multiple_of(a0[p] + kk * RC, 8)
            offs.append(off)
            pltpu.sync_copy(sidx.at[pl.ds(off, RC)], idx_v.at[p])
            pltpu.sync_copy(slbl.at[pl.ds(off, RC)], lbl_v.at[p])
        for p in range(NS):
            pltpu.async_copy(feat.at[idx_v.at[p]], rows_v.at[p], sem)
        for p in range(NS):
            pltpu.make_async_copy(feat.at[idx_v.at[p]], rows_v.at[p],
                                  sem).wait()

        def grp_body(g, carry2):
            lblv = [lbl_v[p, pl.ds(g * 16, 16)] for p in range(NS)]
            for u in range(16):
                i = g * 16 + u
                for p in range(NS):
                    j = k * RC + i
                    valid = jnp.logical_and(j >= skip[p], j < end[p])
                    cc = lax.select(valid,
                                    lblv[p][u] - (base_cls + p * CPS),
                                    jnp.int32(CPS))
                    acc0 = jnp.zeros((16,), jnp.float32)
                    acc1 = jnp.zeros((16,), jnp.float32)
                    rs = []
                    for q in range(NQ):
                        f = rows_v[p, i, pl.ds(q * 16, 16)]
                        pv = protos_v[p][cc, pl.ds(q * 16, 16)]
                        r = pv * EMA + f * ONE_M
                        rs.append(r)
                        if q % 2 == 0:
                            acc0 = acc0 + r * r
                        else:
                            acc1 = acc1 + r * r
                    s = jnp.sum(acc0 + acc1)
                    sv = jnp.full((16,), s, dtype=jnp.float32)
                    y = jnp.minimum(_rsqrt16(sv), 1e12)
                    for q in range(NQ):
                        protos_v[p][cc, pl.ds(q * 16, 16)] = rs[q] * y
            return carry2

        lax.fori_loop(0, RC // 16, grp_body, 0)
        return carry

    lax.fori_loop(0, nch_max, chunk_body, 0)

    for p in range(NS):
        pltpu.sync_copy(protos_v[p].at[pl.ds(0, CPS)],
                        protos_out.at[pl.ds(base_cls + p * CPS, CPS)])


_sc_update = functools.partial(
    pl.kernel,
    mesh=plsc.VectorSubcoreMesh(core_axis_name="c", subcore_axis_name="s"),
    out_type=jax.ShapeDtypeStruct((NCLS_PAD, D), jnp.float32),
    scratch_types=[
        pltpu.VMEM((16,), jnp.int32),
        pltpu.VMEM((NS, RC), jnp.int32),
        pltpu.VMEM((NS, RC), jnp.int32),
        pltpu.VMEM((NS, RC, D), jnp.float32),
        pltpu.VMEM((CPS + 1, D), jnp.float32),
        pltpu.VMEM((CPS + 1, D), jnp.float32),
        pltpu.SemaphoreType.DMA,
    ],
    compiler_params=pltpu.CompilerParams(needs_layout_passes=False),
)(_sc_body)


def _loss_body(protos_ref, out_ref):
    p = protos_ref[...]
    logits = lax.dot_general(p, p, (((1,), (1,)), ((), ())),
                             preferred_element_type=jnp.float32) * INV_T
    row = lax.broadcasted_iota(jnp.int32, (NCLS_PAD, NCLS_PAD), 0)
    col = lax.broadcasted_iota(jnp.int32, (NCLS_PAD, NCLS_PAD), 1)
    mask = jnp.logical_and(row != col,
                           jnp.logical_and(row < N_CLS, col < N_CLS))
    e = jnp.where(mask, jnp.exp(logits), 0.0)
    ssum = jnp.sum(e, axis=1, keepdims=True)
    mpn = jnp.log(ssum * (1.0 / (N_CLS - 1)))
    rvalid = lax.broadcasted_iota(jnp.int32, (NCLS_PAD, 1), 0) < N_CLS
    tot = jnp.sum(jnp.where(rvalid, mpn, 0.0), axis=0, keepdims=True)
    out_ref[...] = tot * (1.0 / N_CLS)


_loss_call = pl.pallas_call(
    _loss_body,
    out_shape=jax.ShapeDtypeStruct((1, 1), jnp.float32),
)


def kernel(features, labels, prototypes):
    labels = labels.astype(jnp.int32)
    nb = labels.shape[0]
    key = (labels << 14) | jnp.arange(nb, dtype=jnp.int32)
    skey = jnp.sort(key)
    order = skey & jnp.int32(0x3FFF)
    slbl = skey >> 14
    counts = jnp.zeros((NCLS_PAD,), jnp.int32).at[labels].add(1)
    starts = jnp.concatenate(
        [jnp.zeros((1,), jnp.int32), jnp.cumsum(counts, dtype=jnp.int32)])

    w = jnp.arange(NW, dtype=jnp.int32)
    sp = starts[(w[:, None] * CPT
                 + jnp.arange(NS + 1, dtype=jnp.int32)[None, :] * CPS)]
    s0 = sp[:, :NS]
    a0 = (s0 // 8) * 8
    skip = s0 - a0
    nrow = sp[:, 1:] - s0
    nch = jnp.maximum((skip + nrow + (RC - 1)) // RC, 1)
    nch_max = jnp.max(nch, axis=1)

    meta = jnp.zeros((NW, 16), jnp.int32)
    meta = (meta.at[:, 0:2].set(a0).at[:, 2:4].set(skip)
                .at[:, 4:6].set(nrow).at[:, 6:8].set(nch)
                .at[:, 8].set(nch_max))

    pad_i = jnp.zeros((HPAD,), jnp.int32)
    sidx_pad = jnp.concatenate([order, pad_i])
    slbl_pad = jnp.concatenate([slbl, pad_i])
    protos_pad = jnp.concatenate(
        [prototypes.astype(jnp.float32),
         jnp.zeros((NCLS_PAD - N_CLS, D), jnp.float32)], axis=0)

    protos_upd = _sc_update(features.astype(jnp.float32), sidx_pad, slbl_pad,
                            protos_pad, meta)
    return _loss_call(protos_upd)[0, 0]
